# Initial kernel scaffold; baseline (speedup 1.0000x reference)
import functools

import jax
import jax.numpy as jnp
from jax import lax
from jax.experimental import pallas as pl
from jax.experimental.pallas import tpu as pltpu
from jax.experimental.pallas import tpu_sc as plsc

N = 10000
E = 320000
HID = 128
OUT_HALF = 64
N_CLASSES = 32
EPS = 1e-5

NC = 2
NS = 16
NW = NC * NS
C = 80
EPW = E // NW
NCHUNK = EPW // C
RPT = N // NS

BLK = 1000
NBLK = N // BLK


def _make_seg(feat, with_deg):
  mesh = plsc.VectorSubcoreMesh(core_axis_name="c", subcore_axis_name="s")
  out_type = [jax.ShapeDtypeStruct((NC * N, feat), jnp.float32)]
  scratch = [
      pltpu.VMEM((NCHUNK, C), jnp.int32),
      pltpu.VMEM((NCHUNK, C), jnp.int32),
      pltpu.VMEM((C, feat), jnp.float32),
      pltpu.VMEM_SHARED((N, feat), jnp.float32),
  ]
  if with_deg:
    out_type.append(jax.ShapeDtypeStruct((NC * N, 16), jnp.float32))
    scratch += [
        pltpu.VMEM((C, 16), jnp.float32),
        pltpu.VMEM_SHARED((N, 16), jnp.float32),
    ]

  if with_deg:
    def body(y, srcr, dstr, zf, z16, ones16, out, outdeg,
             src_v, dst_v, rows_v, acc, one_v, dacc):
      c = lax.axis_index("c")
      s = lax.axis_index("s")
      wid = c * NS + s
      pltpu.sync_copy(zf, acc.at[pl.ds(s * RPT, RPT)])
      pltpu.sync_copy(z16, dacc.at[pl.ds(s * RPT, RPT)])
      pltpu.sync_copy(srcr.at[pl.ds(wid * NCHUNK, NCHUNK)], src_v)
      pltpu.sync_copy(dstr.at[pl.ds(wid * NCHUNK, NCHUNK)], dst_v)
      pltpu.sync_copy(ones16, one_v)
      plsc.subcore_barrier()

      def step(j, carry):
        pltpu.sync_copy(y.at[src_v.at[j]], rows_v)
        pltpu.sync_copy(rows_v, acc.at[dst_v.at[j]], add=True)
        pltpu.sync_copy(one_v, dacc.at[dst_v.at[j]], add=True)
        return carry

      lax.fori_loop(0, NCHUNK, step, 0)
      plsc.subcore_barrier()
      pltpu.sync_copy(acc.at[pl.ds(s * RPT, RPT)],
                      out.at[pl.ds(c * N + s * RPT, RPT)])
      pltpu.sync_copy(dacc.at[pl.ds(s * RPT, RPT)],
                      outdeg.at[pl.ds(c * N + s * RPT, RPT)])
  else:
    def body(y, srcr, dstr, zf, out, src_v, dst_v, rows_v, acc):
      c = lax.axis_index("c")
      s = lax.axis_index("s")
      wid = c * NS + s
      pltpu.sync_copy(zf, acc.at[pl.ds(s * RPT, RPT)])
      pltpu.sync_copy(srcr.at[pl.ds(wid * NCHUNK, NCHUNK)], src_v)
      pltpu.sync_copy(dstr.at[pl.ds(wid * NCHUNK, NCHUNK)], dst_v)
      plsc.subcore_barrier()

      def step(j, carry):
        pltpu.sync_copy(y.at[src_v.at[j]], rows_v)
        pltpu.sync_copy(rows_v, acc.at[dst_v.at[j]], add=True)
        return carry

      lax.fori_loop(0, NCHUNK, step, 0)
      plsc.subcore_barrier()
      pltpu.sync_copy(acc.at[pl.ds(s * RPT, RPT)],
                      out.at[pl.ds(c * N + s * RPT, RPT)])

  return pl.kernel(body, out_type=out_type, mesh=mesh, scratch_types=scratch,
                   name=f"sc_segsum_{feat}{'_deg' if with_deg else ''}")


_seg_deg = _make_seg(HID, True)
_seg_128 = _make_seg(HID, False)
_seg_64 = _make_seg(OUT_HALF, False)


_SQRT_HALF = 0.7071067811865476


def _ln(h, g, b):
  mu = jnp.mean(h, axis=-1, keepdims=True)
  var = jnp.mean((h - mu) ** 2, axis=-1, keepdims=True)
  return (h - mu) * lax.rsqrt(var + EPS) * g + b


def _gelu(h):
  return 0.5 * h * (1.0 + lax.erf(h * _SQRT_HALF))


def _pre_body(x_ref, w_ref, o_ref):
  o_ref[...] = jnp.dot(x_ref[...], w_ref[...],
                       preferred_element_type=jnp.float32)


def _tc_pre(x, w):
  return pl.pallas_call(
      _pre_body,
      grid=(NBLK,),
      in_specs=[
          pl.BlockSpec((BLK, HID), lambda i: (i, 0)),
          pl.BlockSpec((HID, HID), lambda i: (0, 0)),
      ],
      out_specs=pl.BlockSpec((BLK, HID), lambda i: (i, 0)),
      out_shape=jax.ShapeDtypeStruct((N, HID), jnp.float32),
  )(x, w)


def _b12_body(s0, s1, d0, d1, x_ref, wr, bb, g, be, wn, xo, yo):
  inv = 1.0 / jnp.maximum(d0[:, 0:1] + d1[:, 0:1], 1.0)
  h = (s0[...] + s1[...]) * inv + bb[...] + jnp.dot(
      x_ref[...], wr[...], preferred_element_type=jnp.float32)
  h = _gelu(_ln(h, g[...], be[...]))
  xn = h + x_ref[...]
  xo[...] = xn
  yo[...] = jnp.dot(xn, wn[...], preferred_element_type=jnp.float32)


def _tc_block(S, D, x, wr, bb, g, be, wn):
  nxt = wn.shape[1]
  return pl.pallas_call(
      _b12_body,
      grid=(NBLK,),
      in_specs=[
          pl.BlockSpec((BLK, HID), lambda i: (i, 0)),
          pl.BlockSpec((BLK, HID), lambda i: (i + NBLK, 0)),
          pl.BlockSpec((BLK, 16), lambda i: (i, 0)),
          pl.BlockSpec((BLK, 16), lambda i: (i + NBLK, 0)),
          pl.BlockSpec((BLK, HID), lambda i: (i, 0)),
          pl.BlockSpec((HID, HID), lambda i: (0, 0)),
          pl.BlockSpec((1, HID), lambda i: (0, 0)),
          pl.BlockSpec((1, HID), lambda i: (0, 0)),
          pl.BlockSpec((1, HID), lambda i: (0, 0)),
          pl.BlockSpec((HID, nxt), lambda i: (0, 0)),
      ],
      out_specs=[
          pl.BlockSpec((BLK, HID), lambda i: (i, 0)),
          pl.BlockSpec((BLK, nxt), lambda i: (i, 0)),
      ],
      out_shape=[
          jax.ShapeDtypeStruct((N, HID), jnp.float32),
          jax.ShapeDtypeStruct((N, nxt), jnp.float32),
      ],
  )(S, S, D, D, x, wr, bb, g, be, wn)


def _b3_body(s0, s1, d0, d1, x_ref, wr, bb, g, be, wc, bc, o_ref):
  inv = 1.0 / jnp.maximum(d0[:, 0:1] + d1[:, 0:1], 1.0)
  h = (s0[...] + s1[...]) * inv + bb[...] + jnp.dot(
      x_ref[...], wr[...], preferred_element_type=jnp.float32)
  h = _gelu(h)
  h = _ln(h, g[...], be[...])
  o_ref[...] = jnp.dot(h, wc[...], preferred_element_type=jnp.float32) + bc[...]


def _tc_head(S, D, x, wr, bb, g, be, wc, bc):
  return pl.pallas_call(
      _b3_body,
      grid=(NBLK,),
      in_specs=[
          pl.BlockSpec((BLK, OUT_HALF), lambda i: (i, 0)),
          pl.BlockSpec((BLK, OUT_HALF), lambda i: (i + NBLK, 0)),
          pl.BlockSpec((BLK, 16), lambda i: (i, 0)),
          pl.BlockSpec((BLK, 16), lambda i: (i + NBLK, 0)),
          pl.BlockSpec((BLK, HID), lambda i: (i, 0)),
          pl.BlockSpec((HID, OUT_HALF), lambda i: (0, 0)),
          pl.BlockSpec((1, OUT_HALF), lambda i: (0, 0)),
          pl.BlockSpec((1, OUT_HALF), lambda i: (0, 0)),
          pl.BlockSpec((1, OUT_HALF), lambda i: (0, 0)),
          pl.BlockSpec((OUT_HALF, N_CLASSES), lambda i: (0, 0)),
          pl.BlockSpec((1, N_CLASSES), lambda i: (0, 0)),
      ],
      out_specs=pl.BlockSpec((BLK, N_CLASSES), lambda i: (i, 0)),
      out_shape=jax.ShapeDtypeStruct((N, N_CLASSES), jnp.float32),
  )(S, S, D, D, x, wr, bb, g, be, wc, bc)


def kernel(x, edge_index, W1l, b1, W1r, ln1_g, ln1_b, W2l, b2, W2r,
           ln2_g, ln2_b, W3l, b3, W3r, lnc_g, lnc_b, Wc, bc):
  ei = edge_index.astype(jnp.int32)
  srcr = ei[0].reshape(E // C, C)
  dstr = ei[1].reshape(E // C, C)
  zf = jnp.zeros((RPT, HID), jnp.float32)
  z64 = jnp.zeros((RPT, OUT_HALF), jnp.float32)
  z16 = jnp.zeros((RPT, 16), jnp.float32)
  ones16 = jnp.ones((C, 16), jnp.float32)

  r2 = lambda v: v.reshape(1, -1)

  y1 = _tc_pre(x, W1l)
  S1, D = _seg_deg(y1, srcr, dstr, zf, z16, ones16)
  x1, y2 = _tc_block(S1, D, x, W1r, r2(b1), r2(ln1_g), r2(ln1_b), W2l)
  S2 = _seg_128(y2, srcr, dstr, zf)
  x2, y3 = _tc_block(S2, D, x1, W2r, r2(b2), r2(ln2_g), r2(ln2_b), W3l)
  S3 = _seg_64(y3, srcr, dstr, z64)
  return _tc_head(S3, D, x2, W3r, r2(b3), r2(lnc_g), r2(lnc_b), Wc, bc)


# speedup vs baseline: 6.8880x; 6.8880x over previous
import functools

import jax
import jax.numpy as jnp
from jax import lax
from jax.experimental import pallas as pl
from jax.experimental.pallas import tpu as pltpu
from jax.experimental.pallas import tpu_sc as plsc

N = 10000
E = 320000
HID = 128
OUT_HALF = 64
N_CLASSES = 32
EPS = 1e-5

NC = 2
NS = 16
NW = NC * NS
C = 80
EPW = E // NW
NCHUNK = EPW // C
Z0 = 640
ZL = N - (NS - 1) * Z0

BLK = 1000
NBLK = N // BLK


_MESH = plsc.VectorSubcoreMesh(core_axis_name="c", subcore_axis_name="s")


def _zero(s, zsrc, dst_sp):
  @pl.when(s < NS - 1)
  def _():
    pltpu.sync_copy(zsrc, dst_sp.at[pl.ds(s * Z0, Z0)])
  @pl.when(s == NS - 1)
  def _():
    pltpu.sync_copy(zsrc.at[pl.ds(0, ZL)], dst_sp.at[pl.ds((NS - 1) * Z0, ZL)])


def _dump(c, s, src_sp, dst_hbm):
  @pl.when(s < NS - 1)
  def _():
    pltpu.sync_copy(src_sp.at[pl.ds(s * Z0, Z0)],
                    dst_hbm.at[pl.ds(c * N + s * Z0, Z0)])
  @pl.when(s == NS - 1)
  def _():
    pltpu.sync_copy(src_sp.at[pl.ds((NS - 1) * Z0, ZL)],
                    dst_hbm.at[pl.ds(c * N + (NS - 1) * Z0, ZL)])


def _make_seg(feat):
  out_type = jax.ShapeDtypeStruct((NC * N, feat), jnp.float32)
  scratch = [
      pltpu.VMEM((NCHUNK, C), jnp.int32),
      pltpu.VMEM((NCHUNK, C), jnp.int32),
      pltpu.VMEM((C, feat), jnp.float32),
      pltpu.VMEM_SHARED((N, feat), jnp.float32),
  ]

  def body(y, srcr, dstr, zf, out, src_v, dst_v, rows_v, acc):
    c = lax.axis_index("c")
    s = lax.axis_index("s")
    wid = c * NS + s
    _zero(s, zf, acc)
    pltpu.sync_copy(srcr.at[wid], src_v)
    pltpu.sync_copy(dstr.at[wid], dst_v)
    plsc.subcore_barrier()

    def step(j, carry):
      pltpu.sync_copy(y.at[src_v.at[j]], rows_v)
      pltpu.sync_copy(rows_v, acc.at[dst_v.at[j]], add=True)
      return carry

    lax.fori_loop(0, NCHUNK, step, 0)
    plsc.subcore_barrier()
    _dump(c, s, acc, out)

  return pl.kernel(body, out_type=out_type, mesh=_MESH,
                   scratch_types=scratch, name=f"sc_segsum_{feat}")


def _deg_body(dstr, onesf, zf, outdeg, dst_v, one_v, dacc):
  c = lax.axis_index("c")
  s = lax.axis_index("s")
  wid = c * NS + s
  _zero(s, zf, dacc)
  pltpu.sync_copy(dstr.at[wid], dst_v)
  pltpu.sync_copy(onesf, one_v)
  plsc.subcore_barrier()

  def step(j, carry):
    pltpu.sync_copy(one_v, dacc.at[dst_v.at[j]], add=True)
    return carry

  lax.fori_loop(0, NCHUNK, step, 0)
  plsc.subcore_barrier()
  _dump(c, s, dacc, outdeg)


_seg_deg = pl.kernel(
    _deg_body,
    out_type=jax.ShapeDtypeStruct((NC * N, HID), jnp.float32),
    mesh=_MESH,
    scratch_types=[
        pltpu.VMEM((NCHUNK, C), jnp.int32),
        pltpu.VMEM((C, HID), jnp.float32),
        pltpu.VMEM_SHARED((N, HID), jnp.float32),
    ],
    name="sc_degree")

_seg_128 = _make_seg(HID)


_SQRT_HALF = 0.7071067811865476


def _ln(h, g, b):
  mu = jnp.mean(h, axis=-1, keepdims=True)
  var = jnp.mean((h - mu) ** 2, axis=-1, keepdims=True)
  return (h - mu) * lax.rsqrt(var + EPS) * g + b


def _gelu(h):
  return 0.5 * h * (1.0 + lax.erf(h * _SQRT_HALF))


def _pre_body(x_ref, w_ref, o_ref):
  o_ref[...] = jnp.dot(x_ref[...], w_ref[...],
                       preferred_element_type=jnp.float32)


def _tc_pre(x, w):
  return pl.pallas_call(
      _pre_body,
      grid=(NBLK,),
      in_specs=[
          pl.BlockSpec((BLK, HID), lambda i: (i, 0)),
          pl.BlockSpec((HID, HID), lambda i: (0, 0)),
      ],
      out_specs=pl.BlockSpec((BLK, HID), lambda i: (i, 0)),
      out_shape=jax.ShapeDtypeStruct((N, HID), jnp.float32),
  )(x, w)


def _deg_reduce_body(d0, d1, o_ref):
  o_ref[...] = 1.0 / jnp.maximum(d0[:, 0:1] + d1[:, 0:1], 1.0)


def _tc_deg_reduce(D):
  return pl.pallas_call(
      _deg_reduce_body,
      grid=(NBLK,),
      in_specs=[
          pl.BlockSpec((BLK, HID), lambda i: (i, 0)),
          pl.BlockSpec((BLK, HID), lambda i: (i + NBLK, 0)),
      ],
      out_specs=pl.BlockSpec((BLK, 1), lambda i: (i, 0)),
      out_shape=jax.ShapeDtypeStruct((N, 1), jnp.float32),
  )(D, D)


def _b12_body(s0, s1, inv_ref, x_ref, wr, bb, g, be, wn, xo, yo):
  inv = inv_ref[...]
  h = (s0[...] + s1[...]) * inv + bb[...] + jnp.dot(
      x_ref[...], wr[...], preferred_element_type=jnp.float32)
  h = _gelu(_ln(h, g[...], be[...]))
  xn = h + x_ref[...]
  xo[...] = xn
  yo[...] = jnp.dot(xn, wn[...], preferred_element_type=jnp.float32)


def _tc_block(S, invd, x, wr, bb, g, be, wn):
  nxt = wn.shape[1]
  return pl.pallas_call(
      _b12_body,
      grid=(NBLK,),
      in_specs=[
          pl.BlockSpec((BLK, HID), lambda i: (i, 0)),
          pl.BlockSpec((BLK, HID), lambda i: (i + NBLK, 0)),
          pl.BlockSpec((BLK, 1), lambda i: (i, 0)),
          pl.BlockSpec((BLK, HID), lambda i: (i, 0)),
          pl.BlockSpec((HID, HID), lambda i: (0, 0)),
          pl.BlockSpec((1, HID), lambda i: (0, 0)),
          pl.BlockSpec((1, HID), lambda i: (0, 0)),
          pl.BlockSpec((1, HID), lambda i: (0, 0)),
          pl.BlockSpec((HID, nxt), lambda i: (0, 0)),
      ],
      out_specs=[
          pl.BlockSpec((BLK, HID), lambda i: (i, 0)),
          pl.BlockSpec((BLK, nxt), lambda i: (i, 0)),
      ],
      out_shape=[
          jax.ShapeDtypeStruct((N, HID), jnp.float32),
          jax.ShapeDtypeStruct((N, nxt), jnp.float32),
      ],
  )(S, S, invd, x, wr, bb, g, be, wn)


def _b3_body(s0, s1, inv_ref, x_ref, wr, bb, g, be, wc, bc, o_ref):
  inv = inv_ref[...]
  h = (s0[:, :OUT_HALF] + s1[:, :OUT_HALF]) * inv + bb[...] + jnp.dot(
      x_ref[...], wr[...], preferred_element_type=jnp.float32)
  h = _gelu(h)
  h = _ln(h, g[...], be[...])
  o_ref[...] = jnp.dot(h, wc[...], preferred_element_type=jnp.float32) + bc[...]


def _tc_head(S, invd, x, wr, bb, g, be, wc, bc):
  return pl.pallas_call(
      _b3_body,
      grid=(NBLK,),
      in_specs=[
          pl.BlockSpec((BLK, HID), lambda i: (i, 0)),
          pl.BlockSpec((BLK, HID), lambda i: (i + NBLK, 0)),
          pl.BlockSpec((BLK, 1), lambda i: (i, 0)),
          pl.BlockSpec((BLK, HID), lambda i: (i, 0)),
          pl.BlockSpec((HID, OUT_HALF), lambda i: (0, 0)),
          pl.BlockSpec((1, OUT_HALF), lambda i: (0, 0)),
          pl.BlockSpec((1, OUT_HALF), lambda i: (0, 0)),
          pl.BlockSpec((1, OUT_HALF), lambda i: (0, 0)),
          pl.BlockSpec((OUT_HALF, N_CLASSES), lambda i: (0, 0)),
          pl.BlockSpec((1, N_CLASSES), lambda i: (0, 0)),
      ],
      out_specs=pl.BlockSpec((BLK, N_CLASSES), lambda i: (i, 0)),
      out_shape=jax.ShapeDtypeStruct((N, N_CLASSES), jnp.float32),
  )(S, S, invd, x, wr, bb, g, be, wc, bc)


def kernel(x, edge_index, W1l, b1, W1r, ln1_g, ln1_b, W2l, b2, W2r,
           ln2_g, ln2_b, W3l, b3, W3r, lnc_g, lnc_b, Wc, bc):
  ei = edge_index.astype(jnp.int32)
  srcr = ei[0].reshape(NW, NCHUNK, C)
  dstr = ei[1].reshape(NW, NCHUNK, C)
  zf = jnp.zeros((Z0, HID), jnp.float32)
  onesf = jnp.ones((C, HID), jnp.float32)
  W3lp = jnp.concatenate(
      [W3l, jnp.zeros((HID, HID - OUT_HALF), jnp.float32)], axis=1)

  r2 = lambda v: v.reshape(1, -1)

  y1 = _tc_pre(x, W1l)
  D = _seg_deg(dstr, onesf, zf)
  invd = _tc_deg_reduce(D)
  S1 = _seg_128(y1, srcr, dstr, zf)
  x1, y2 = _tc_block(S1, invd, x, W1r, r2(b1), r2(ln1_g), r2(ln1_b), W2l)
  S2 = _seg_128(y2, srcr, dstr, zf)
  x2, y3 = _tc_block(S2, invd, x1, W2r, r2(b2), r2(ln2_g), r2(ln2_b), W3lp)
  S3 = _seg_128(y3, srcr, dstr, zf)
  return _tc_head(S3, invd, x2, W3r, r2(b3), r2(lnc_g), r2(lnc_b),
                  Wc, r2(bc))


# speedup vs baseline: 8.5782x; 1.2454x over previous
import functools

import jax
import jax.numpy as jnp
from jax import lax
from jax.experimental import pallas as pl
from jax.experimental.pallas import tpu as pltpu
from jax.experimental.pallas import tpu_sc as plsc

N = 10000
E = 320000
HID = 128
OUT_HALF = 64
N_CLASSES = 32
EPS = 1e-5

NC = 2
NS = 16
NW = NC * NS
C = 80
EPW = E // NW
NCHUNK = EPW // C
Z0 = 640
ZL = N - (NS - 1) * Z0

BLK = 1000
NBLK = N // BLK


_MESH = plsc.VectorSubcoreMesh(core_axis_name="c", subcore_axis_name="s")


def _zero(s, zsrc, dst_sp):
  @pl.when(s < NS - 1)
  def _():
    pltpu.sync_copy(zsrc, dst_sp.at[pl.ds(s * Z0, Z0)])
  @pl.when(s == NS - 1)
  def _():
    pltpu.sync_copy(zsrc.at[pl.ds(0, ZL)], dst_sp.at[pl.ds((NS - 1) * Z0, ZL)])


def _dump(c, s, src_sp, dst_hbm):
  @pl.when(s < NS - 1)
  def _():
    pltpu.sync_copy(src_sp.at[pl.ds(s * Z0, Z0)],
                    dst_hbm.at[pl.ds(c * N + s * Z0, Z0)])
  @pl.when(s == NS - 1)
  def _():
    pltpu.sync_copy(src_sp.at[pl.ds((NS - 1) * Z0, ZL)],
                    dst_hbm.at[pl.ds(c * N + (NS - 1) * Z0, ZL)])


def _make_seg(feat):
  out_type = jax.ShapeDtypeStruct((NC * N, feat), jnp.float32)
  scratch = [
      pltpu.VMEM((EPW,), jnp.int32),
      pltpu.VMEM((NCHUNK, C), jnp.int32),
      pltpu.VMEM((C, feat), jnp.float32),
      pltpu.VMEM((C, feat), jnp.float32),
      pltpu.VMEM_SHARED((N, feat), jnp.float32),
      pltpu.SemaphoreType.DMA,
      pltpu.SemaphoreType.DMA,
  ]

  def body(y, srcf, dstr, zf, out, src_v, dst_v, rows0, rows1, acc,
           sem0, sem1):
    c = lax.axis_index("c")
    s = lax.axis_index("s")
    wid = c * NS + s
    _zero(s, zf, acc)
    pltpu.sync_copy(srcf.at[wid], src_v)
    pltpu.sync_copy(dstr.at[wid], dst_v)
    plsc.subcore_barrier()

    def _idx(j):
      return src_v.at[pl.ds(pl.multiple_of(j * C, 8), C)]

    def gat_start(j, buf, sem):
      pltpu.async_copy(y.at[_idx(j)], buf, sem)

    def gat_wait(j, buf, sem):
      pltpu.make_async_copy(y.at[_idx(j)], buf, sem).wait()

    gat_start(0, rows0, sem0)

    def step(k, carry):
      a = 2 * k
      gat_wait(a, rows0, sem0)
      gat_start(a + 1, rows1, sem1)
      pltpu.sync_copy(rows0, acc.at[dst_v.at[a]], add=True)
      gat_wait(a + 1, rows1, sem1)
      gat_start(a + 2, rows0, sem0)
      pltpu.sync_copy(rows1, acc.at[dst_v.at[a + 1]], add=True)
      return carry

    lax.fori_loop(0, (NCHUNK - 1) // 2, step, 0)
    gat_wait(NCHUNK - 1, rows0, sem0)
    pltpu.sync_copy(rows0, acc.at[dst_v.at[NCHUNK - 1]], add=True)
    plsc.subcore_barrier()
    _dump(c, s, acc, out)

  return pl.kernel(body, out_type=out_type, mesh=_MESH,
                   scratch_types=scratch, name=f"sc_segsum_{feat}")


def _deg_body(dstr, onesf, zf, outdeg, dst_v, one_v, dacc):
  c = lax.axis_index("c")
  s = lax.axis_index("s")
  wid = c * NS + s
  _zero(s, zf, dacc)
  pltpu.sync_copy(dstr.at[wid], dst_v)
  pltpu.sync_copy(onesf, one_v)
  plsc.subcore_barrier()

  def step(j, carry):
    pltpu.sync_copy(one_v, dacc.at[dst_v.at[j]], add=True)
    return carry

  lax.fori_loop(0, NCHUNK, step, 0)
  plsc.subcore_barrier()
  _dump(c, s, dacc, outdeg)


_seg_deg = pl.kernel(
    _deg_body,
    out_type=jax.ShapeDtypeStruct((NC * N, HID), jnp.float32),
    mesh=_MESH,
    scratch_types=[
        pltpu.VMEM((NCHUNK, C), jnp.int32),
        pltpu.VMEM((C, HID), jnp.float32),
        pltpu.VMEM_SHARED((N, HID), jnp.float32),
    ],
    name="sc_degree")

_seg_128 = _make_seg(HID)


_SQRT_HALF = 0.7071067811865476


def _ln(h, g, b):
  mu = jnp.mean(h, axis=-1, keepdims=True)
  var = jnp.mean((h - mu) ** 2, axis=-1, keepdims=True)
  return (h - mu) * lax.rsqrt(var + EPS) * g + b


def _gelu(h):
  return 0.5 * h * (1.0 + lax.erf(h * _SQRT_HALF))


def _pre_body(x_ref, w_ref, o_ref):
  o_ref[...] = jnp.dot(x_ref[...], w_ref[...],
                       preferred_element_type=jnp.float32)


def _tc_pre(x, w):
  return pl.pallas_call(
      _pre_body,
      grid=(NBLK,),
      in_specs=[
          pl.BlockSpec((BLK, HID), lambda i: (i, 0)),
          pl.BlockSpec((HID, HID), lambda i: (0, 0)),
      ],
      out_specs=pl.BlockSpec((BLK, HID), lambda i: (i, 0)),
      out_shape=jax.ShapeDtypeStruct((N, HID), jnp.float32),
  )(x, w)


def _deg_reduce_body(d0, d1, o_ref):
  o_ref[...] = 1.0 / jnp.maximum(d0[:, 0:1] + d1[:, 0:1], 1.0)


def _tc_deg_reduce(D):
  return pl.pallas_call(
      _deg_reduce_body,
      grid=(NBLK,),
      in_specs=[
          pl.BlockSpec((BLK, HID), lambda i: (i, 0)),
          pl.BlockSpec((BLK, HID), lambda i: (i + NBLK, 0)),
      ],
      out_specs=pl.BlockSpec((BLK, 1), lambda i: (i, 0)),
      out_shape=jax.ShapeDtypeStruct((N, 1), jnp.float32),
  )(D, D)


def _b12_body(s0, s1, inv_ref, x_ref, wr, bb, g, be, wn, xo, yo):
  inv = inv_ref[...]
  h = (s0[...] + s1[...]) * inv + bb[...] + jnp.dot(
      x_ref[...], wr[...], preferred_element_type=jnp.float32)
  h = _gelu(_ln(h, g[...], be[...]))
  xn = h + x_ref[...]
  xo[...] = xn
  yo[...] = jnp.dot(xn, wn[...], preferred_element_type=jnp.float32)


def _tc_block(S, invd, x, wr, bb, g, be, wn):
  nxt = wn.shape[1]
  return pl.pallas_call(
      _b12_body,
      grid=(NBLK,),
      in_specs=[
          pl.BlockSpec((BLK, HID), lambda i: (i, 0)),
          pl.BlockSpec((BLK, HID), lambda i: (i + NBLK, 0)),
          pl.BlockSpec((BLK, 1), lambda i: (i, 0)),
          pl.BlockSpec((BLK, HID), lambda i: (i, 0)),
          pl.BlockSpec((HID, HID), lambda i: (0, 0)),
          pl.BlockSpec((1, HID), lambda i: (0, 0)),
          pl.BlockSpec((1, HID), lambda i: (0, 0)),
          pl.BlockSpec((1, HID), lambda i: (0, 0)),
          pl.BlockSpec((HID, nxt), lambda i: (0, 0)),
      ],
      out_specs=[
          pl.BlockSpec((BLK, HID), lambda i: (i, 0)),
          pl.BlockSpec((BLK, nxt), lambda i: (i, 0)),
      ],
      out_shape=[
          jax.ShapeDtypeStruct((N, HID), jnp.float32),
          jax.ShapeDtypeStruct((N, nxt), jnp.float32),
      ],
  )(S, S, invd, x, wr, bb, g, be, wn)


def _b3_body(s0, s1, inv_ref, x_ref, wr, bb, g, be, wc, bc, o_ref):
  inv = inv_ref[...]
  h = (s0[:, :OUT_HALF] + s1[:, :OUT_HALF]) * inv + bb[...] + jnp.dot(
      x_ref[...], wr[...], preferred_element_type=jnp.float32)
  h = _gelu(h)
  h = _ln(h, g[...], be[...])
  o_ref[...] = jnp.dot(h, wc[...], preferred_element_type=jnp.float32) + bc[...]


def _tc_head(S, invd, x, wr, bb, g, be, wc, bc):
  return pl.pallas_call(
      _b3_body,
      grid=(NBLK,),
      in_specs=[
          pl.BlockSpec((BLK, HID), lambda i: (i, 0)),
          pl.BlockSpec((BLK, HID), lambda i: (i + NBLK, 0)),
          pl.BlockSpec((BLK, 1), lambda i: (i, 0)),
          pl.BlockSpec((BLK, HID), lambda i: (i, 0)),
          pl.BlockSpec((HID, OUT_HALF), lambda i: (0, 0)),
          pl.BlockSpec((1, OUT_HALF), lambda i: (0, 0)),
          pl.BlockSpec((1, OUT_HALF), lambda i: (0, 0)),
          pl.BlockSpec((1, OUT_HALF), lambda i: (0, 0)),
          pl.BlockSpec((OUT_HALF, N_CLASSES), lambda i: (0, 0)),
          pl.BlockSpec((1, N_CLASSES), lambda i: (0, 0)),
      ],
      out_specs=pl.BlockSpec((BLK, N_CLASSES), lambda i: (i, 0)),
      out_shape=jax.ShapeDtypeStruct((N, N_CLASSES), jnp.float32),
  )(S, S, invd, x, wr, bb, g, be, wc, bc)


def kernel(x, edge_index, W1l, b1, W1r, ln1_g, ln1_b, W2l, b2, W2r,
           ln2_g, ln2_b, W3l, b3, W3r, lnc_g, lnc_b, Wc, bc):
  ei = edge_index.astype(jnp.int32)
  srcf = ei[0].reshape(NW, EPW)
  dstr = ei[1].reshape(NW, NCHUNK, C)
  zf = jnp.zeros((Z0, HID), jnp.float32)
  onesf = jnp.ones((C, HID), jnp.float32)
  W3lp = jnp.concatenate(
      [W3l, jnp.zeros((HID, HID - OUT_HALF), jnp.float32)], axis=1)

  r2 = lambda v: v.reshape(1, -1)

  y1 = _tc_pre(x, W1l)
  D = _seg_deg(dstr, onesf, zf)
  invd = _tc_deg_reduce(D)
  S1 = _seg_128(y1, srcf, dstr, zf)
  x1, y2 = _tc_block(S1, invd, x, W1r, r2(b1), r2(ln1_g), r2(ln1_b), W2l)
  S2 = _seg_128(y2, srcf, dstr, zf)
  x2, y3 = _tc_block(S2, invd, x1, W2r, r2(b2), r2(ln2_g), r2(ln2_b), W3lp)
  S3 = _seg_128(y3, srcf, dstr, zf)
  return _tc_head(S3, invd, x2, W3r, r2(b3), r2(lnc_g), r2(lnc_b),
                  Wc, r2(bc))


# speedup vs baseline: 8.6416x; 1.0074x over previous
import functools

import jax
import jax.numpy as jnp
from jax import lax
from jax.experimental import pallas as pl
from jax.experimental.pallas import tpu as pltpu
from jax.experimental.pallas import tpu_sc as plsc

N = 10000
E = 320000
HID = 128
OUT_HALF = 64
N_CLASSES = 32
EPS = 1e-5

NC = 2
NS = 16
NW = NC * NS
C = 80
EPW = E // NW
NCHUNK = EPW // C
Z0 = 640
ZL = N - (NS - 1) * Z0

BLK = 1000
NBLK = N // BLK


_MESH = plsc.VectorSubcoreMesh(core_axis_name="c", subcore_axis_name="s")


def _zero(s, zsrc, dst_sp):
  @pl.when(s < NS - 1)
  def _():
    pltpu.sync_copy(zsrc, dst_sp.at[pl.ds(s * Z0, Z0)])
  @pl.when(s == NS - 1)
  def _():
    pltpu.sync_copy(zsrc.at[pl.ds(0, ZL)], dst_sp.at[pl.ds((NS - 1) * Z0, ZL)])


def _dump(c, s, src_sp, dst_hbm):
  @pl.when(s < NS - 1)
  def _():
    pltpu.sync_copy(src_sp.at[pl.ds(s * Z0, Z0)],
                    dst_hbm.at[pl.ds(c * N + s * Z0, Z0)])
  @pl.when(s == NS - 1)
  def _():
    pltpu.sync_copy(src_sp.at[pl.ds((NS - 1) * Z0, ZL)],
                    dst_hbm.at[pl.ds(c * N + (NS - 1) * Z0, ZL)])


def _make_seg(feat):
  out_type = jax.ShapeDtypeStruct((NC * N, feat), jnp.float32)
  scratch = [
      pltpu.VMEM((EPW,), jnp.int32),
      pltpu.VMEM((NCHUNK, C), jnp.int32),
      pltpu.VMEM((C, feat), jnp.float32),
      pltpu.VMEM((C, feat), jnp.float32),
      pltpu.VMEM_SHARED((N, feat), jnp.float32),
      pltpu.SemaphoreType.DMA,
      pltpu.SemaphoreType.DMA,
      pltpu.SemaphoreType.DMA,
      pltpu.SemaphoreType.DMA,
  ]

  def body(y, srcf, dstr, zf, out, src_v, dst_v, rows0, rows1, acc,
           sem0, sem1, sems0, sems1):
    c = lax.axis_index("c")
    s = lax.axis_index("s")
    wid = c * NS + s
    _zero(s, zf, acc)
    pltpu.sync_copy(srcf.at[wid], src_v)
    pltpu.sync_copy(dstr.at[wid], dst_v)
    plsc.subcore_barrier()

    def _idx(j):
      return src_v.at[pl.ds(pl.multiple_of(j * C, 8), C)]

    def gat_start(j, buf, sem):
      pltpu.async_copy(y.at[_idx(j)], buf, sem)

    def gat_wait(j, buf, sem):
      pltpu.make_async_copy(y.at[_idx(j)], buf, sem).wait()

    def sca_start(j, buf, sem):
      pltpu.async_copy(buf, acc.at[dst_v.at[j]], sem, add=True)

    def sca_wait(j, buf, sem):
      pltpu.make_async_copy(buf, acc.at[dst_v.at[j]], sem).wait()

    gat_start(0, rows0, sem0)
    gat_start(1, rows1, sem1)

    def step(k, carry):
      a = 2 * k
      gat_wait(a, rows0, sem0)
      sca_start(a, rows0, sems0)

      @pl.when(a + 1 < NCHUNK)
      def _():
        gat_wait(a + 1, rows1, sem1)
        sca_start(a + 1, rows1, sems1)

      @pl.when(a + 2 < NCHUNK)
      def _():
        sca_wait(a, rows0, sems0)
        gat_start(a + 2, rows0, sem0)

      @pl.when(a + 3 < NCHUNK)
      def _():
        sca_wait(a + 1, rows1, sems1)
        gat_start(a + 3, rows1, sem1)

      return carry

    lax.fori_loop(0, (NCHUNK + 1) // 2, step, 0)
    sca_wait(NCHUNK - 2, rows1, sems1)
    sca_wait(NCHUNK - 1, rows0, sems0)
    plsc.subcore_barrier()
    _dump(c, s, acc, out)

  return pl.kernel(body, out_type=out_type, mesh=_MESH,
                   scratch_types=scratch, name=f"sc_segsum_{feat}")


def _deg_body(dstr, onesf, zf, outdeg, dst_v, one_v, dacc, sems0, sems1):
  c = lax.axis_index("c")
  s = lax.axis_index("s")
  wid = c * NS + s
  _zero(s, zf, dacc)
  pltpu.sync_copy(dstr.at[wid], dst_v)
  pltpu.sync_copy(onesf, one_v)
  plsc.subcore_barrier()

  def sca_start(j, sem):
    pltpu.async_copy(one_v, dacc.at[dst_v.at[j]], sem, add=True)

  def sca_wait(j, sem):
    pltpu.make_async_copy(one_v, dacc.at[dst_v.at[j]], sem).wait()

  sca_start(0, sems0)
  sca_start(1, sems1)

  def step(k, carry):
    a = 2 * k
    sca_wait(a - 2, sems0)
    sca_start(a, sems0)

    @pl.when(a + 1 < NCHUNK)
    def _():
      sca_wait(a - 1, sems1)
      sca_start(a + 1, sems1)

    return carry

  lax.fori_loop(1, (NCHUNK + 1) // 2, step, 0)
  sca_wait(NCHUNK - 2, sems1)
  sca_wait(NCHUNK - 1, sems0)
  plsc.subcore_barrier()
  _dump(c, s, dacc, outdeg)


_seg_deg = pl.kernel(
    _deg_body,
    out_type=jax.ShapeDtypeStruct((NC * N, HID), jnp.float32),
    mesh=_MESH,
    scratch_types=[
        pltpu.VMEM((NCHUNK, C), jnp.int32),
        pltpu.VMEM((C, HID), jnp.float32),
        pltpu.VMEM_SHARED((N, HID), jnp.float32),
        pltpu.SemaphoreType.DMA,
        pltpu.SemaphoreType.DMA,
    ],
    name="sc_degree")

_seg_128 = _make_seg(HID)


_SQRT_HALF = 0.7071067811865476


def _ln(h, g, b):
  mu = jnp.mean(h, axis=-1, keepdims=True)
  var = jnp.mean((h - mu) ** 2, axis=-1, keepdims=True)
  return (h - mu) * lax.rsqrt(var + EPS) * g + b


def _gelu(h):
  return 0.5 * h * (1.0 + lax.erf(h * _SQRT_HALF))


def _pre_body(x_ref, w_ref, o_ref):
  o_ref[...] = jnp.dot(x_ref[...], w_ref[...],
                       preferred_element_type=jnp.float32)


def _tc_pre(x, w):
  return pl.pallas_call(
      _pre_body,
      grid=(NBLK,),
      in_specs=[
          pl.BlockSpec((BLK, HID), lambda i: (i, 0)),
          pl.BlockSpec((HID, HID), lambda i: (0, 0)),
      ],
      out_specs=pl.BlockSpec((BLK, HID), lambda i: (i, 0)),
      out_shape=jax.ShapeDtypeStruct((N, HID), jnp.float32),
  )(x, w)


def _deg_reduce_body(d0, d1, o_ref):
  o_ref[...] = 1.0 / jnp.maximum(d0[:, 0:1] + d1[:, 0:1], 1.0)


def _tc_deg_reduce(D):
  return pl.pallas_call(
      _deg_reduce_body,
      grid=(NBLK,),
      in_specs=[
          pl.BlockSpec((BLK, HID), lambda i: (i, 0)),
          pl.BlockSpec((BLK, HID), lambda i: (i + NBLK, 0)),
      ],
      out_specs=pl.BlockSpec((BLK, 1), lambda i: (i, 0)),
      out_shape=jax.ShapeDtypeStruct((N, 1), jnp.float32),
  )(D, D)


def _b12_body(s0, s1, inv_ref, x_ref, wr, bb, g, be, wn, xo, yo):
  inv = inv_ref[...]
  h = (s0[...] + s1[...]) * inv + bb[...] + jnp.dot(
      x_ref[...], wr[...], preferred_element_type=jnp.float32)
  h = _gelu(_ln(h, g[...], be[...]))
  xn = h + x_ref[...]
  xo[...] = xn
  yo[...] = jnp.dot(xn, wn[...], preferred_element_type=jnp.float32)


def _tc_block(S, invd, x, wr, bb, g, be, wn):
  nxt = wn.shape[1]
  return pl.pallas_call(
      _b12_body,
      grid=(NBLK,),
      in_specs=[
          pl.BlockSpec((BLK, HID), lambda i: (i, 0)),
          pl.BlockSpec((BLK, HID), lambda i: (i + NBLK, 0)),
          pl.BlockSpec((BLK, 1), lambda i: (i, 0)),
          pl.BlockSpec((BLK, HID), lambda i: (i, 0)),
          pl.BlockSpec((HID, HID), lambda i: (0, 0)),
          pl.BlockSpec((1, HID), lambda i: (0, 0)),
          pl.BlockSpec((1, HID), lambda i: (0, 0)),
          pl.BlockSpec((1, HID), lambda i: (0, 0)),
          pl.BlockSpec((HID, nxt), lambda i: (0, 0)),
      ],
      out_specs=[
          pl.BlockSpec((BLK, HID), lambda i: (i, 0)),
          pl.BlockSpec((BLK, nxt), lambda i: (i, 0)),
      ],
      out_shape=[
          jax.ShapeDtypeStruct((N, HID), jnp.float32),
          jax.ShapeDtypeStruct((N, nxt), jnp.float32),
      ],
  )(S, S, invd, x, wr, bb, g, be, wn)


def _b3_body(s0, s1, inv_ref, x_ref, wr, bb, g, be, wc, bc, o_ref):
  inv = inv_ref[...]
  h = (s0[:, :OUT_HALF] + s1[:, :OUT_HALF]) * inv + bb[...] + jnp.dot(
      x_ref[...], wr[...], preferred_element_type=jnp.float32)
  h = _gelu(h)
  h = _ln(h, g[...], be[...])
  o_ref[...] = jnp.dot(h, wc[...], preferred_element_type=jnp.float32) + bc[...]


def _tc_head(S, invd, x, wr, bb, g, be, wc, bc):
  return pl.pallas_call(
      _b3_body,
      grid=(NBLK,),
      in_specs=[
          pl.BlockSpec((BLK, HID), lambda i: (i, 0)),
          pl.BlockSpec((BLK, HID), lambda i: (i + NBLK, 0)),
          pl.BlockSpec((BLK, 1), lambda i: (i, 0)),
          pl.BlockSpec((BLK, HID), lambda i: (i, 0)),
          pl.BlockSpec((HID, OUT_HALF), lambda i: (0, 0)),
          pl.BlockSpec((1, OUT_HALF), lambda i: (0, 0)),
          pl.BlockSpec((1, OUT_HALF), lambda i: (0, 0)),
          pl.BlockSpec((1, OUT_HALF), lambda i: (0, 0)),
          pl.BlockSpec((OUT_HALF, N_CLASSES), lambda i: (0, 0)),
          pl.BlockSpec((1, N_CLASSES), lambda i: (0, 0)),
      ],
      out_specs=pl.BlockSpec((BLK, N_CLASSES), lambda i: (i, 0)),
      out_shape=jax.ShapeDtypeStruct((N, N_CLASSES), jnp.float32),
  )(S, S, invd, x, wr, bb, g, be, wc, bc)


def kernel(x, edge_index, W1l, b1, W1r, ln1_g, ln1_b, W2l, b2, W2r,
           ln2_g, ln2_b, W3l, b3, W3r, lnc_g, lnc_b, Wc, bc):
  ei = edge_index.astype(jnp.int32)
  srcf = ei[0].reshape(NW, EPW)
  dstr = ei[1].reshape(NW, NCHUNK, C)
  zf = jnp.zeros((Z0, HID), jnp.float32)
  onesf = jnp.ones((C, HID), jnp.float32)
  W3lp = jnp.concatenate(
      [W3l, jnp.zeros((HID, HID - OUT_HALF), jnp.float32)], axis=1)

  r2 = lambda v: v.reshape(1, -1)

  y1 = _tc_pre(x, W1l)
  D = _seg_deg(dstr, onesf, zf)
  invd = _tc_deg_reduce(D)
  S1 = _seg_128(y1, srcf, dstr, zf)
  x1, y2 = _tc_block(S1, invd, x, W1r, r2(b1), r2(ln1_g), r2(ln1_b), W2l)
  S2 = _seg_128(y2, srcf, dstr, zf)
  x2, y3 = _tc_block(S2, invd, x1, W2r, r2(b2), r2(ln2_g), r2(ln2_b), W3lp)
  S3 = _seg_128(y3, srcf, dstr, zf)
  return _tc_head(S3, invd, x2, W3r, r2(b3), r2(lnc_g), r2(lnc_b),
                  Wc, r2(bc))


# speedup vs baseline: 8.6719x; 1.0035x over previous
import functools

import jax
import jax.numpy as jnp
from jax import lax
from jax.experimental import pallas as pl
from jax.experimental.pallas import tpu as pltpu
from jax.experimental.pallas import tpu_sc as plsc

N = 10000
E = 320000
HID = 128
OUT_HALF = 64
N_CLASSES = 32
EPS = 1e-5

NC = 2
NS = 16
NW = NC * NS
C = 80
EPW = E // NW
NCHUNK = EPW // C
Z0 = 640
ZL = N - (NS - 1) * Z0

BLK = 2000
NBLK = N // BLK


_MESH = plsc.VectorSubcoreMesh(core_axis_name="c", subcore_axis_name="s")


def _zero(s, zsrc, dst_sp):
  @pl.when(s < NS - 1)
  def _():
    pltpu.sync_copy(zsrc, dst_sp.at[pl.ds(s * Z0, Z0)])
  @pl.when(s == NS - 1)
  def _():
    pltpu.sync_copy(zsrc.at[pl.ds(0, ZL)], dst_sp.at[pl.ds((NS - 1) * Z0, ZL)])


def _dump(c, s, src_sp, dst_hbm):
  @pl.when(s < NS - 1)
  def _():
    pltpu.sync_copy(src_sp.at[pl.ds(s * Z0, Z0)],
                    dst_hbm.at[pl.ds(c * N + s * Z0, Z0)])
  @pl.when(s == NS - 1)
  def _():
    pltpu.sync_copy(src_sp.at[pl.ds((NS - 1) * Z0, ZL)],
                    dst_hbm.at[pl.ds(c * N + (NS - 1) * Z0, ZL)])


def _make_seg(feat):
  out_type = jax.ShapeDtypeStruct((NC * N, feat), jnp.float32)
  scratch = [
      pltpu.VMEM((EPW,), jnp.int32),
      pltpu.VMEM((NCHUNK, C), jnp.int32),
      pltpu.VMEM((C, feat), jnp.float32),
      pltpu.VMEM((C, feat), jnp.float32),
      pltpu.VMEM_SHARED((N, feat), jnp.float32),
      pltpu.SemaphoreType.DMA,
      pltpu.SemaphoreType.DMA,
      pltpu.SemaphoreType.DMA,
      pltpu.SemaphoreType.DMA,
  ]

  def body(y, srcf, dstr, zf, out, src_v, dst_v, rows0, rows1, acc,
           sem0, sem1, sems0, sems1):
    c = lax.axis_index("c")
    s = lax.axis_index("s")
    wid = c * NS + s
    _zero(s, zf, acc)
    pltpu.sync_copy(srcf.at[wid], src_v)
    pltpu.sync_copy(dstr.at[wid], dst_v)
    plsc.subcore_barrier()

    def _idx(j):
      return src_v.at[pl.ds(pl.multiple_of(j * C, 8), C)]

    def gat_start(j, buf, sem):
      pltpu.async_copy(y.at[_idx(j)], buf, sem)

    def gat_wait(j, buf, sem):
      pltpu.make_async_copy(y.at[_idx(j)], buf, sem).wait()

    def sca_start(j, buf, sem):
      pltpu.async_copy(buf, acc.at[dst_v.at[j]], sem, add=True)

    def sca_wait(j, buf, sem):
      pltpu.make_async_copy(buf, acc.at[dst_v.at[j]], sem).wait()

    gat_start(0, rows0, sem0)
    gat_start(1, rows1, sem1)

    def step(k, carry):
      a = 2 * k
      gat_wait(a, rows0, sem0)
      sca_start(a, rows0, sems0)

      @pl.when(a + 1 < NCHUNK)
      def _():
        gat_wait(a + 1, rows1, sem1)
        sca_start(a + 1, rows1, sems1)

      @pl.when(a + 2 < NCHUNK)
      def _():
        sca_wait(a, rows0, sems0)
        gat_start(a + 2, rows0, sem0)

      @pl.when(a + 3 < NCHUNK)
      def _():
        sca_wait(a + 1, rows1, sems1)
        gat_start(a + 3, rows1, sem1)

      return carry

    lax.fori_loop(0, (NCHUNK + 1) // 2, step, 0)
    sca_wait(NCHUNK - 2, rows1, sems1)
    sca_wait(NCHUNK - 1, rows0, sems0)
    plsc.subcore_barrier()
    _dump(c, s, acc, out)

  return pl.kernel(body, out_type=out_type, mesh=_MESH,
                   scratch_types=scratch, name=f"sc_segsum_{feat}")


def _deg_body(dstr, onesf, zf, outdeg, dst_v, one_v, dacc, sems0, sems1):
  c = lax.axis_index("c")
  s = lax.axis_index("s")
  wid = c * NS + s
  _zero(s, zf, dacc)
  pltpu.sync_copy(dstr.at[wid], dst_v)
  pltpu.sync_copy(onesf, one_v)
  plsc.subcore_barrier()

  def sca_start(j, sem):
    pltpu.async_copy(one_v, dacc.at[dst_v.at[j]], sem, add=True)

  def sca_wait(j, sem):
    pltpu.make_async_copy(one_v, dacc.at[dst_v.at[j]], sem).wait()

  sca_start(0, sems0)
  sca_start(1, sems1)

  def step(k, carry):
    a = 2 * k
    sca_wait(a - 2, sems0)
    sca_start(a, sems0)

    @pl.when(a + 1 < NCHUNK)
    def _():
      sca_wait(a - 1, sems1)
      sca_start(a + 1, sems1)

    return carry

  lax.fori_loop(1, (NCHUNK + 1) // 2, step, 0)
  sca_wait(NCHUNK - 2, sems1)
  sca_wait(NCHUNK - 1, sems0)
  plsc.subcore_barrier()
  _dump(c, s, dacc, outdeg)


_seg_deg = pl.kernel(
    _deg_body,
    out_type=jax.ShapeDtypeStruct((NC * N, HID), jnp.float32),
    mesh=_MESH,
    scratch_types=[
        pltpu.VMEM((NCHUNK, C), jnp.int32),
        pltpu.VMEM((C, HID), jnp.float32),
        pltpu.VMEM_SHARED((N, HID), jnp.float32),
        pltpu.SemaphoreType.DMA,
        pltpu.SemaphoreType.DMA,
    ],
    name="sc_degree")

_seg_128 = _make_seg(HID)


_SQRT_HALF = 0.7071067811865476


def _ln(h, g, b):
  mu = jnp.mean(h, axis=-1, keepdims=True)
  var = jnp.mean((h - mu) ** 2, axis=-1, keepdims=True)
  return (h - mu) * lax.rsqrt(var + EPS) * g + b


def _gelu(h):
  return 0.5 * h * (1.0 + lax.erf(h * _SQRT_HALF))


def _pre_body(x_ref, w_ref, o_ref):
  o_ref[...] = jnp.dot(x_ref[...], w_ref[...],
                       preferred_element_type=jnp.float32)


def _tc_pre(x, w):
  return pl.pallas_call(
      _pre_body,
      grid=(NBLK,),
      in_specs=[
          pl.BlockSpec((BLK, HID), lambda i: (i, 0)),
          pl.BlockSpec((HID, HID), lambda i: (0, 0)),
      ],
      out_specs=pl.BlockSpec((BLK, HID), lambda i: (i, 0)),
      out_shape=jax.ShapeDtypeStruct((N, HID), jnp.float32),
  )(x, w)


def _b12_body(s0, s1, d0, d1, x_ref, wr, bb, g, be, wn, xo, yo):
  inv = 1.0 / jnp.maximum(d0[:, 0:1] + d1[:, 0:1], 1.0)
  h = (s0[...] + s1[...]) * inv + bb[...] + jnp.dot(
      x_ref[...], wr[...], preferred_element_type=jnp.float32)
  h = _gelu(_ln(h, g[...], be[...]))
  xn = h + x_ref[...]
  xo[...] = xn
  yo[...] = jnp.dot(xn, wn[...], preferred_element_type=jnp.float32)


def _tc_block(S, D, x, wr, bb, g, be, wn):
  nxt = wn.shape[1]
  return pl.pallas_call(
      _b12_body,
      grid=(NBLK,),
      in_specs=[
          pl.BlockSpec((BLK, HID), lambda i: (i, 0)),
          pl.BlockSpec((BLK, HID), lambda i: (i + NBLK, 0)),
          pl.BlockSpec((BLK, HID), lambda i: (i, 0)),
          pl.BlockSpec((BLK, HID), lambda i: (i + NBLK, 0)),
          pl.BlockSpec((BLK, HID), lambda i: (i, 0)),
          pl.BlockSpec((HID, HID), lambda i: (0, 0)),
          pl.BlockSpec((1, HID), lambda i: (0, 0)),
          pl.BlockSpec((1, HID), lambda i: (0, 0)),
          pl.BlockSpec((1, HID), lambda i: (0, 0)),
          pl.BlockSpec((HID, nxt), lambda i: (0, 0)),
      ],
      out_specs=[
          pl.BlockSpec((BLK, HID), lambda i: (i, 0)),
          pl.BlockSpec((BLK, nxt), lambda i: (i, 0)),
      ],
      out_shape=[
          jax.ShapeDtypeStruct((N, HID), jnp.float32),
          jax.ShapeDtypeStruct((N, nxt), jnp.float32),
      ],
  )(S, S, D, D, x, wr, bb, g, be, wn)


def _b3_body(s0, s1, d0, d1, x_ref, wr, bb, g, be, wc, bc, o_ref):
  inv = 1.0 / jnp.maximum(d0[:, 0:1] + d1[:, 0:1], 1.0)
  h = (s0[:, :OUT_HALF] + s1[:, :OUT_HALF]) * inv + bb[...] + jnp.dot(
      x_ref[...], wr[...], preferred_element_type=jnp.float32)
  h = _gelu(h)
  h = _ln(h, g[...], be[...])
  o_ref[...] = jnp.dot(h, wc[...], preferred_element_type=jnp.float32) + bc[...]


def _tc_head(S, D, x, wr, bb, g, be, wc, bc):
  return pl.pallas_call(
      _b3_body,
      grid=(NBLK,),
      in_specs=[
          pl.BlockSpec((BLK, HID), lambda i: (i, 0)),
          pl.BlockSpec((BLK, HID), lambda i: (i + NBLK, 0)),
          pl.BlockSpec((BLK, HID), lambda i: (i, 0)),
          pl.BlockSpec((BLK, HID), lambda i: (i + NBLK, 0)),
          pl.BlockSpec((BLK, HID), lambda i: (i, 0)),
          pl.BlockSpec((HID, OUT_HALF), lambda i: (0, 0)),
          pl.BlockSpec((1, OUT_HALF), lambda i: (0, 0)),
          pl.BlockSpec((1, OUT_HALF), lambda i: (0, 0)),
          pl.BlockSpec((1, OUT_HALF), lambda i: (0, 0)),
          pl.BlockSpec((OUT_HALF, N_CLASSES), lambda i: (0, 0)),
          pl.BlockSpec((1, N_CLASSES), lambda i: (0, 0)),
      ],
      out_specs=pl.BlockSpec((BLK, N_CLASSES), lambda i: (i, 0)),
      out_shape=jax.ShapeDtypeStruct((N, N_CLASSES), jnp.float32),
  )(S, S, D, D, x, wr, bb, g, be, wc, bc)


def kernel(x, edge_index, W1l, b1, W1r, ln1_g, ln1_b, W2l, b2, W2r,
           ln2_g, ln2_b, W3l, b3, W3r, lnc_g, lnc_b, Wc, bc):
  ei = edge_index.astype(jnp.int32)
  srcf = ei[0].reshape(NW, EPW)
  dstr = ei[1].reshape(NW, NCHUNK, C)
  zf = jnp.zeros((Z0, HID), jnp.float32)
  onesf = jnp.ones((C, HID), jnp.float32)
  W3lp = jnp.concatenate(
      [W3l, jnp.zeros((HID, HID - OUT_HALF), jnp.float32)], axis=1)

  r2 = lambda v: v.reshape(1, -1)

  D = _seg_deg(dstr, onesf, zf)
  y1 = _tc_pre(x, W1l)
  S1 = _seg_128(y1, srcf, dstr, zf)
  x1, y2 = _tc_block(S1, D, x, W1r, r2(b1), r2(ln1_g), r2(ln1_b), W2l)
  S2 = _seg_128(y2, srcf, dstr, zf)
  x2, y3 = _tc_block(S2, D, x1, W2r, r2(b2), r2(ln2_g), r2(ln2_b), W3lp)
  S3 = _seg_128(y3, srcf, dstr, zf)
  return _tc_head(S3, D, x2, W3r, r2(b3), r2(lnc_g), r2(lnc_b),
                  Wc, r2(bc))


# speedup vs baseline: 8.9440x; 1.0314x over previous
import functools

import jax
import jax.numpy as jnp
from jax import lax
from jax.experimental import pallas as pl
from jax.experimental.pallas import tpu as pltpu
from jax.experimental.pallas import tpu_sc as plsc

N = 10000
E = 320000
HID = 128
OUT_HALF = 64
N_CLASSES = 32
EPS = 1e-5

NC = 2
NS = 16
NW = NC * NS
C = 80
EPW = E // NW
NCHUNK = EPW // C
Z0 = 640
ZL = N - (NS - 1) * Z0

BLK = 2000
NBLK = N // BLK


_MESH = plsc.VectorSubcoreMesh(core_axis_name="c", subcore_axis_name="s")


def _zero(s, zsrc, dst_sp):
  @pl.when(s < NS - 1)
  def _():
    pltpu.sync_copy(zsrc, dst_sp.at[pl.ds(s * Z0, Z0)])
  @pl.when(s == NS - 1)
  def _():
    pltpu.sync_copy(zsrc.at[pl.ds(0, ZL)], dst_sp.at[pl.ds((NS - 1) * Z0, ZL)])


def _dump(c, s, src_sp, dst_hbm):
  @pl.when(s < NS - 1)
  def _():
    pltpu.sync_copy(src_sp.at[pl.ds(s * Z0, Z0)],
                    dst_hbm.at[pl.ds(c * N + s * Z0, Z0)])
  @pl.when(s == NS - 1)
  def _():
    pltpu.sync_copy(src_sp.at[pl.ds((NS - 1) * Z0, ZL)],
                    dst_hbm.at[pl.ds(c * N + (NS - 1) * Z0, ZL)])


def _make_seg(feat):
  out_type = jax.ShapeDtypeStruct((NC * N, feat), jnp.float32)
  scratch = [
      pltpu.VMEM((EPW,), jnp.int32),
      pltpu.VMEM((NCHUNK, C), jnp.int32),
      pltpu.VMEM((C, feat), jnp.float32),
      pltpu.VMEM((C, feat), jnp.float32),
      pltpu.VMEM_SHARED((N, feat), jnp.float32),
      pltpu.SemaphoreType.DMA,
      pltpu.SemaphoreType.DMA,
      pltpu.SemaphoreType.DMA,
      pltpu.SemaphoreType.DMA,
  ]

  def body(y, srcf, dstr, zf, out, src_v, dst_v, rows0, rows1, acc,
           sem0, sem1, sems0, sems1):
    c = lax.axis_index("c")
    s = lax.axis_index("s")
    wid = c * NS + s
    _zero(s, zf, acc)
    pltpu.sync_copy(srcf.at[wid], src_v)
    pltpu.sync_copy(dstr.at[wid], dst_v)
    plsc.subcore_barrier()

    def _idx(j):
      return src_v.at[pl.ds(pl.multiple_of(j * C, 8), C)]

    def gat_start(j, buf, sem):
      pltpu.async_copy(y.at[_idx(j)], buf, sem)

    def gat_wait(j, buf, sem):
      pltpu.make_async_copy(y.at[_idx(j)], buf, sem).wait()

    def sca_start(j, buf, sem):
      pltpu.async_copy(buf, acc.at[dst_v.at[j]], sem, add=True)

    def sca_wait(j, buf, sem):
      pltpu.make_async_copy(buf, acc.at[dst_v.at[j]], sem).wait()

    gat_start(0, rows0, sem0)
    gat_start(1, rows1, sem1)

    def step(k, carry):
      a = 2 * k
      gat_wait(a, rows0, sem0)
      sca_start(a, rows0, sems0)

      @pl.when(a + 1 < NCHUNK)
      def _():
        gat_wait(a + 1, rows1, sem1)
        sca_start(a + 1, rows1, sems1)

      @pl.when(a + 2 < NCHUNK)
      def _():
        sca_wait(a, rows0, sems0)
        gat_start(a + 2, rows0, sem0)

      @pl.when(a + 3 < NCHUNK)
      def _():
        sca_wait(a + 1, rows1, sems1)
        gat_start(a + 3, rows1, sem1)

      return carry

    lax.fori_loop(0, (NCHUNK + 1) // 2, step, 0)
    sca_wait(NCHUNK - 2, rows1, sems1)
    sca_wait(NCHUNK - 1, rows0, sems0)
    plsc.subcore_barrier()
    _dump(c, s, acc, out)

  return pl.kernel(body, out_type=out_type, mesh=_MESH,
                   scratch_types=scratch, name=f"sc_segsum_{feat}")


def _seg1_deg_body(y, srcf, dstr, zf, onesf, out, outdeg,
                   src_v, dst_v, rows0, rows1, acc, sem0, sem1, sems0, sems1):
  c = lax.axis_index("c")
  s = lax.axis_index("s")
  wid = c * NS + s
  _zero(s, zf, acc)
  pltpu.sync_copy(srcf.at[wid], src_v)
  pltpu.sync_copy(dstr.at[wid], dst_v)
  pltpu.sync_copy(onesf, rows0)
  plsc.subcore_barrier()

  def dsca_start(j, sem):
    pltpu.async_copy(rows0, acc.at[dst_v.at[j]], sem, add=True)

  def dsca_wait(j, sem):
    pltpu.make_async_copy(rows0, acc.at[dst_v.at[j]], sem).wait()

  dsca_start(0, sems0)
  dsca_start(1, sems1)

  def dstep(k, carry):
    a = 2 * k
    dsca_wait(a - 2, sems0)
    dsca_start(a, sems0)

    @pl.when(a + 1 < NCHUNK)
    def _():
      dsca_wait(a - 1, sems1)
      dsca_start(a + 1, sems1)

    return carry

  lax.fori_loop(1, (NCHUNK + 1) // 2, dstep, 0)
  dsca_wait(NCHUNK - 2, sems1)
  dsca_wait(NCHUNK - 1, sems0)
  plsc.subcore_barrier()
  _dump(c, s, acc, outdeg)
  _zero(s, zf, acc)
  plsc.subcore_barrier()

  def _idx(j):
    return src_v.at[pl.ds(pl.multiple_of(j * C, 8), C)]

  def gat_start(j, buf, sem):
    pltpu.async_copy(y.at[_idx(j)], buf, sem)

  def gat_wait(j, buf, sem):
    pltpu.make_async_copy(y.at[_idx(j)], buf, sem).wait()

  def sca_start(j, buf, sem):
    pltpu.async_copy(buf, acc.at[dst_v.at[j]], sem, add=True)

  def sca_wait(j, buf, sem):
    pltpu.make_async_copy(buf, acc.at[dst_v.at[j]], sem).wait()

  gat_start(0, rows0, sem0)
  gat_start(1, rows1, sem1)

  def step(k, carry):
    a = 2 * k
    gat_wait(a, rows0, sem0)
    sca_start(a, rows0, sems0)

    @pl.when(a + 1 < NCHUNK)
    def _():
      gat_wait(a + 1, rows1, sem1)
      sca_start(a + 1, rows1, sems1)

    @pl.when(a + 2 < NCHUNK)
    def _():
      sca_wait(a, rows0, sems0)
      gat_start(a + 2, rows0, sem0)

    @pl.when(a + 3 < NCHUNK)
    def _():
      sca_wait(a + 1, rows1, sems1)
      gat_start(a + 3, rows1, sem1)

    return carry

  lax.fori_loop(0, (NCHUNK + 1) // 2, step, 0)
  sca_wait(NCHUNK - 2, rows1, sems1)
  sca_wait(NCHUNK - 1, rows0, sems0)
  plsc.subcore_barrier()
  _dump(c, s, acc, out)


_seg1_deg = pl.kernel(
    _seg1_deg_body,
    out_type=[jax.ShapeDtypeStruct((NC * N, HID), jnp.float32),
              jax.ShapeDtypeStruct((NC * N, HID), jnp.float32)],
    mesh=_MESH,
    scratch_types=[
        pltpu.VMEM((EPW,), jnp.int32),
        pltpu.VMEM((NCHUNK, C), jnp.int32),
        pltpu.VMEM((C, HID), jnp.float32),
        pltpu.VMEM((C, HID), jnp.float32),
        pltpu.VMEM_SHARED((N, HID), jnp.float32),
        pltpu.SemaphoreType.DMA,
        pltpu.SemaphoreType.DMA,
        pltpu.SemaphoreType.DMA,
        pltpu.SemaphoreType.DMA,
    ],
    name="sc_seg1_deg")

_seg_128 = _make_seg(HID)


_SQRT_HALF = 0.7071067811865476


def _ln(h, g, b):
  mu = jnp.mean(h, axis=-1, keepdims=True)
  var = jnp.mean((h - mu) ** 2, axis=-1, keepdims=True)
  return (h - mu) * lax.rsqrt(var + EPS) * g + b


def _gelu(h):
  return 0.5 * h * (1.0 + lax.erf(h * _SQRT_HALF))


def _pre_body(x_ref, w_ref, o_ref):
  o_ref[...] = jnp.dot(x_ref[...], w_ref[...],
                       preferred_element_type=jnp.float32)


def _tc_pre(x, w):
  return pl.pallas_call(
      _pre_body,
      grid=(NBLK,),
      in_specs=[
          pl.BlockSpec((BLK, HID), lambda i: (i, 0)),
          pl.BlockSpec((HID, HID), lambda i: (0, 0)),
      ],
      out_specs=pl.BlockSpec((BLK, HID), lambda i: (i, 0)),
      out_shape=jax.ShapeDtypeStruct((N, HID), jnp.float32),
  )(x, w)


def _b1_body(s0, s1, d0, d1, x_ref, wl, wr, bb, g, be, wn, xo, yo):
  inv = 1.0 / jnp.maximum(d0[:, 0:1] + d1[:, 0:1], 1.0)
  agg = (s0[...] + s1[...]) * inv
  h = jnp.dot(agg, wl[...], preferred_element_type=jnp.float32) + bb[...] \
      + jnp.dot(x_ref[...], wr[...], preferred_element_type=jnp.float32)
  h = _gelu(_ln(h, g[...], be[...]))
  xn = h + x_ref[...]
  xo[...] = xn
  yo[...] = jnp.dot(xn, wn[...], preferred_element_type=jnp.float32)


def _tc_block1(S, D, x, wl, wr, bb, g, be, wn):
  nxt = wn.shape[1]
  return pl.pallas_call(
      _b1_body,
      grid=(NBLK,),
      in_specs=[
          pl.BlockSpec((BLK, HID), lambda i: (i, 0)),
          pl.BlockSpec((BLK, HID), lambda i: (i + NBLK, 0)),
          pl.BlockSpec((BLK, HID), lambda i: (i, 0)),
          pl.BlockSpec((BLK, HID), lambda i: (i + NBLK, 0)),
          pl.BlockSpec((BLK, HID), lambda i: (i, 0)),
          pl.BlockSpec((HID, HID), lambda i: (0, 0)),
          pl.BlockSpec((HID, HID), lambda i: (0, 0)),
          pl.BlockSpec((1, HID), lambda i: (0, 0)),
          pl.BlockSpec((1, HID), lambda i: (0, 0)),
          pl.BlockSpec((1, HID), lambda i: (0, 0)),
          pl.BlockSpec((HID, nxt), lambda i: (0, 0)),
      ],
      out_specs=[
          pl.BlockSpec((BLK, HID), lambda i: (i, 0)),
          pl.BlockSpec((BLK, nxt), lambda i: (i, 0)),
      ],
      out_shape=[
          jax.ShapeDtypeStruct((N, HID), jnp.float32),
          jax.ShapeDtypeStruct((N, nxt), jnp.float32),
      ],
  )(S, S, D, D, x, wl, wr, bb, g, be, wn)


def _b12_body(s0, s1, d0, d1, x_ref, wr, bb, g, be, wn, xo, yo):
  inv = 1.0 / jnp.maximum(d0[:, 0:1] + d1[:, 0:1], 1.0)
  h = (s0[...] + s1[...]) * inv + bb[...] + jnp.dot(
      x_ref[...], wr[...], preferred_element_type=jnp.float32)
  h = _gelu(_ln(h, g[...], be[...]))
  xn = h + x_ref[...]
  xo[...] = xn
  yo[...] = jnp.dot(xn, wn[...], preferred_element_type=jnp.float32)


def _tc_block(S, D, x, wr, bb, g, be, wn):
  nxt = wn.shape[1]
  return pl.pallas_call(
      _b12_body,
      grid=(NBLK,),
      in_specs=[
          pl.BlockSpec((BLK, HID), lambda i: (i, 0)),
          pl.BlockSpec((BLK, HID), lambda i: (i + NBLK, 0)),
          pl.BlockSpec((BLK, HID), lambda i: (i, 0)),
          pl.BlockSpec((BLK, HID), lambda i: (i + NBLK, 0)),
          pl.BlockSpec((BLK, HID), lambda i: (i, 0)),
          pl.BlockSpec((HID, HID), lambda i: (0, 0)),
          pl.BlockSpec((1, HID), lambda i: (0, 0)),
          pl.BlockSpec((1, HID), lambda i: (0, 0)),
          pl.BlockSpec((1, HID), lambda i: (0, 0)),
          pl.BlockSpec((HID, nxt), lambda i: (0, 0)),
      ],
      out_specs=[
          pl.BlockSpec((BLK, HID), lambda i: (i, 0)),
          pl.BlockSpec((BLK, nxt), lambda i: (i, 0)),
      ],
      out_shape=[
          jax.ShapeDtypeStruct((N, HID), jnp.float32),
          jax.ShapeDtypeStruct((N, nxt), jnp.float32),
      ],
  )(S, S, D, D, x, wr, bb, g, be, wn)


def _b3_body(s0, s1, d0, d1, x_ref, wr, bb, g, be, wc, bc, o_ref):
  inv = 1.0 / jnp.maximum(d0[:, 0:1] + d1[:, 0:1], 1.0)
  h = (s0[:, :OUT_HALF] + s1[:, :OUT_HALF]) * inv + bb[...] + jnp.dot(
      x_ref[...], wr[...], preferred_element_type=jnp.float32)
  h = _gelu(h)
  h = _ln(h, g[...], be[...])
  o_ref[...] = jnp.dot(h, wc[...], preferred_element_type=jnp.float32) + bc[...]


def _tc_head(S, D, x, wr, bb, g, be, wc, bc):
  return pl.pallas_call(
      _b3_body,
      grid=(NBLK,),
      in_specs=[
          pl.BlockSpec((BLK, HID), lambda i: (i, 0)),
          pl.BlockSpec((BLK, HID), lambda i: (i + NBLK, 0)),
          pl.BlockSpec((BLK, HID), lambda i: (i, 0)),
          pl.BlockSpec((BLK, HID), lambda i: (i + NBLK, 0)),
          pl.BlockSpec((BLK, HID), lambda i: (i, 0)),
          pl.BlockSpec((HID, OUT_HALF), lambda i: (0, 0)),
          pl.BlockSpec((1, OUT_HALF), lambda i: (0, 0)),
          pl.BlockSpec((1, OUT_HALF), lambda i: (0, 0)),
          pl.BlockSpec((1, OUT_HALF), lambda i: (0, 0)),
          pl.BlockSpec((OUT_HALF, N_CLASSES), lambda i: (0, 0)),
          pl.BlockSpec((1, N_CLASSES), lambda i: (0, 0)),
      ],
      out_specs=pl.BlockSpec((BLK, N_CLASSES), lambda i: (i, 0)),
      out_shape=jax.ShapeDtypeStruct((N, N_CLASSES), jnp.float32),
  )(S, S, D, D, x, wr, bb, g, be, wc, bc)


def kernel(x, edge_index, W1l, b1, W1r, ln1_g, ln1_b, W2l, b2, W2r,
           ln2_g, ln2_b, W3l, b3, W3r, lnc_g, lnc_b, Wc, bc):
  ei = edge_index.astype(jnp.int32)
  srcf = ei[0].reshape(NW, EPW)
  dstr = ei[1].reshape(NW, NCHUNK, C)
  zf = jnp.zeros((Z0, HID), jnp.float32)
  onesf = jnp.ones((C, HID), jnp.float32)
  W3lp = jnp.concatenate(
      [W3l, jnp.zeros((HID, HID - OUT_HALF), jnp.float32)], axis=1)

  r2 = lambda v: v.reshape(1, -1)

  S1, D = _seg1_deg(x, srcf, dstr, zf, onesf)
  x1, y2 = _tc_block1(S1, D, x, W1l, W1r, r2(b1), r2(ln1_g), r2(ln1_b), W2l)
  S2 = _seg_128(y2, srcf, dstr, zf)
  x2, y3 = _tc_block(S2, D, x1, W2r, r2(b2), r2(ln2_g), r2(ln2_b), W3lp)
  S3 = _seg_128(y3, srcf, dstr, zf)
  return _tc_head(S3, D, x2, W3r, r2(b3), r2(lnc_g), r2(lnc_b),
                  Wc, r2(bc))


# speedup vs baseline: 8.9483x; 1.0005x over previous
import functools

import jax
import jax.numpy as jnp
from jax import lax
from jax.experimental import pallas as pl
from jax.experimental.pallas import tpu as pltpu
from jax.experimental.pallas import tpu_sc as plsc

N = 10000
E = 320000
HID = 128
OUT_HALF = 64
N_CLASSES = 32
EPS = 1e-5

NC = 2
NS = 16
NW = NC * NS
C = 80
EPW = E // NW
NCHUNK = EPW // C
Z0 = 640
ZL = N - (NS - 1) * Z0

BLK = 2000
NBLK = N // BLK


_MESH = plsc.VectorSubcoreMesh(core_axis_name="c", subcore_axis_name="s")


def _zero(s, zsrc, dst_sp):
  @pl.when(s < NS - 1)
  def _():
    pltpu.sync_copy(zsrc, dst_sp.at[pl.ds(s * Z0, Z0)])
  @pl.when(s == NS - 1)
  def _():
    pltpu.sync_copy(zsrc.at[pl.ds(0, ZL)], dst_sp.at[pl.ds((NS - 1) * Z0, ZL)])


def _dump(c, s, src_sp, dst_hbm):
  @pl.when(s < NS - 1)
  def _():
    pltpu.sync_copy(src_sp.at[pl.ds(s * Z0, Z0)],
                    dst_hbm.at[pl.ds(c * N + s * Z0, Z0)])
  @pl.when(s == NS - 1)
  def _():
    pltpu.sync_copy(src_sp.at[pl.ds((NS - 1) * Z0, ZL)],
                    dst_hbm.at[pl.ds(c * N + (NS - 1) * Z0, ZL)])


def _make_seg(feat):
  out_type = jax.ShapeDtypeStruct((NC * N, feat), jnp.float32)
  scratch = [
      pltpu.VMEM((EPW,), jnp.int32),
      pltpu.VMEM((NCHUNK, C), jnp.int32),
      pltpu.VMEM((C, feat), jnp.float32),
      pltpu.VMEM((C, feat), jnp.float32),
      pltpu.VMEM_SHARED((N, feat), jnp.float32),
      pltpu.SemaphoreType.DMA,
      pltpu.SemaphoreType.DMA,
      pltpu.SemaphoreType.DMA,
      pltpu.SemaphoreType.DMA,
  ]

  def body(y, srcf, dstr, zf, out, src_v, dst_v, rows0, rows1, acc,
           sem0, sem1, sems0, sems1):
    c = lax.axis_index("c")
    s = lax.axis_index("s")
    wid = c * NS + s
    _zero(s, zf, acc)
    pltpu.sync_copy(srcf.at[wid], src_v)
    pltpu.sync_copy(dstr.at[wid], dst_v)
    plsc.subcore_barrier()

    def _idx(j):
      return src_v.at[pl.ds(pl.multiple_of(j * C, 8), C)]

    def gat_start(j, buf, sem):
      pltpu.async_copy(y.at[_idx(j)], buf, sem)

    def gat_wait(j, buf, sem):
      pltpu.make_async_copy(y.at[_idx(j)], buf, sem).wait()

    def sca_start(j, buf, sem):
      pltpu.async_copy(buf, acc.at[dst_v.at[j]], sem, add=True)

    def sca_wait(j, buf, sem):
      pltpu.make_async_copy(buf, acc.at[dst_v.at[j]], sem).wait()

    gat_start(0, rows0, sem0)
    gat_start(1, rows1, sem1)

    def step(k, carry):
      a = 2 * k
      gat_wait(a, rows0, sem0)
      sca_start(a, rows0, sems0)

      @pl.when(a + 1 < NCHUNK)
      def _():
        gat_wait(a + 1, rows1, sem1)
        sca_start(a + 1, rows1, sems1)

      @pl.when(a + 2 < NCHUNK)
      def _():
        sca_wait(a, rows0, sems0)
        gat_start(a + 2, rows0, sem0)

      @pl.when(a + 3 < NCHUNK)
      def _():
        sca_wait(a + 1, rows1, sems1)
        gat_start(a + 3, rows1, sem1)

      return carry

    lax.fori_loop(0, (NCHUNK + 1) // 2, step, 0)
    sca_wait(NCHUNK - 2, rows1, sems1)
    sca_wait(NCHUNK - 1, rows0, sems0)
    plsc.subcore_barrier()
    _dump(c, s, acc, out)

  return pl.kernel(body, out_type=out_type, mesh=_MESH,
                   scratch_types=scratch, name=f"sc_segsum_{feat}")


def _seg1_deg_body(y, srcf, dstr, zf, onesf, out, outdeg,
                   src_v, dst_v, rows0, rows1, acc, sem0, sem1, sems0, sems1):
  c = lax.axis_index("c")
  s = lax.axis_index("s")
  wid = c * NS + s
  _zero(s, zf, acc)
  pltpu.sync_copy(srcf.at[wid], src_v)
  pltpu.sync_copy(dstr.at[wid], dst_v)
  pltpu.sync_copy(onesf, rows0)
  plsc.subcore_barrier()

  def dsca_start(j, sem):
    pltpu.async_copy(rows0, acc.at[dst_v.at[j]], sem, add=True)

  def dsca_wait(j, sem):
    pltpu.make_async_copy(rows0, acc.at[dst_v.at[j]], sem).wait()

  dsca_start(0, sems0)
  dsca_start(1, sems1)

  def dstep(k, carry):
    a = 2 * k
    dsca_wait(a - 2, sems0)
    dsca_start(a, sems0)

    @pl.when(a + 1 < NCHUNK)
    def _():
      dsca_wait(a - 1, sems1)
      dsca_start(a + 1, sems1)

    return carry

  lax.fori_loop(1, (NCHUNK + 1) // 2, dstep, 0)
  dsca_wait(NCHUNK - 2, sems1)
  dsca_wait(NCHUNK - 1, sems0)
  plsc.subcore_barrier()
  _dump(c, s, acc, outdeg)
  _zero(s, zf, acc)
  plsc.subcore_barrier()

  def _idx(j):
    return src_v.at[pl.ds(pl.multiple_of(j * C, 8), C)]

  def gat_start(j, buf, sem):
    pltpu.async_copy(y.at[_idx(j)], buf, sem)

  def gat_wait(j, buf, sem):
    pltpu.make_async_copy(y.at[_idx(j)], buf, sem).wait()

  def sca_start(j, buf, sem):
    pltpu.async_copy(buf, acc.at[dst_v.at[j]], sem, add=True)

  def sca_wait(j, buf, sem):
    pltpu.make_async_copy(buf, acc.at[dst_v.at[j]], sem).wait()

  gat_start(0, rows0, sem0)
  gat_start(1, rows1, sem1)

  def step(k, carry):
    a = 2 * k
    gat_wait(a, rows0, sem0)
    sca_start(a, rows0, sems0)

    @pl.when(a + 1 < NCHUNK)
    def _():
      gat_wait(a + 1, rows1, sem1)
      sca_start(a + 1, rows1, sems1)

    @pl.when(a + 2 < NCHUNK)
    def _():
      sca_wait(a, rows0, sems0)
      gat_start(a + 2, rows0, sem0)

    @pl.when(a + 3 < NCHUNK)
    def _():
      sca_wait(a + 1, rows1, sems1)
      gat_start(a + 3, rows1, sem1)

    return carry

  lax.fori_loop(0, (NCHUNK + 1) // 2, step, 0)
  sca_wait(NCHUNK - 2, rows1, sems1)
  sca_wait(NCHUNK - 1, rows0, sems0)
  plsc.subcore_barrier()
  _dump(c, s, acc, out)


_seg1_deg = pl.kernel(
    _seg1_deg_body,
    out_type=[jax.ShapeDtypeStruct((NC * N, HID), jnp.float32),
              jax.ShapeDtypeStruct((NC * N, HID), jnp.float32)],
    mesh=_MESH,
    scratch_types=[
        pltpu.VMEM((EPW,), jnp.int32),
        pltpu.VMEM((NCHUNK, C), jnp.int32),
        pltpu.VMEM((C, HID), jnp.float32),
        pltpu.VMEM((C, HID), jnp.float32),
        pltpu.VMEM_SHARED((N, HID), jnp.float32),
        pltpu.SemaphoreType.DMA,
        pltpu.SemaphoreType.DMA,
        pltpu.SemaphoreType.DMA,
        pltpu.SemaphoreType.DMA,
    ],
    name="sc_seg1_deg")

_seg_128 = _make_seg(HID)


_SQRT_HALF = 0.7071067811865476


def _ln(h, g, b):
  mu = jnp.mean(h, axis=-1, keepdims=True)
  var = jnp.mean((h - mu) ** 2, axis=-1, keepdims=True)
  return (h - mu) * lax.rsqrt(var + EPS) * g + b


def _gelu(h):
  return 0.5 * h * (1.0 + lax.erf(h * _SQRT_HALF))


def _pre_body(x_ref, w_ref, o_ref):
  o_ref[...] = jnp.dot(x_ref[...], w_ref[...],
                       preferred_element_type=jnp.float32)


def _tc_pre(x, w):
  return pl.pallas_call(
      _pre_body,
      grid=(NBLK,),
      in_specs=[
          pl.BlockSpec((BLK, HID), lambda i: (i, 0)),
          pl.BlockSpec((HID, HID), lambda i: (0, 0)),
      ],
      out_specs=pl.BlockSpec((BLK, HID), lambda i: (i, 0)),
      out_shape=jax.ShapeDtypeStruct((N, HID), jnp.float32),
  )(x, w)


def _b1_body(s0, s1, d0, d1, x_ref, wl, wr, bb, g, be, wn, xo, yo, invo):
  inv = 1.0 / jnp.maximum(d0[:, 0:1] + d1[:, 0:1], 1.0)
  invo[...] = inv
  agg = (s0[...] + s1[...]) * inv
  h = jnp.dot(agg, wl[...], preferred_element_type=jnp.float32) + bb[...] \
      + jnp.dot(x_ref[...], wr[...], preferred_element_type=jnp.float32)
  h = _gelu(_ln(h, g[...], be[...]))
  xn = h + x_ref[...]
  xo[...] = xn
  yo[...] = jnp.dot(xn, wn[...], preferred_element_type=jnp.float32)


def _tc_block1(S, D, x, wl, wr, bb, g, be, wn):
  nxt = wn.shape[1]
  return pl.pallas_call(
      _b1_body,
      grid=(NBLK,),
      in_specs=[
          pl.BlockSpec((BLK, HID), lambda i: (i, 0)),
          pl.BlockSpec((BLK, HID), lambda i: (i + NBLK, 0)),
          pl.BlockSpec((BLK, HID), lambda i: (i, 0)),
          pl.BlockSpec((BLK, HID), lambda i: (i + NBLK, 0)),
          pl.BlockSpec((BLK, HID), lambda i: (i, 0)),
          pl.BlockSpec((HID, HID), lambda i: (0, 0)),
          pl.BlockSpec((HID, HID), lambda i: (0, 0)),
          pl.BlockSpec((1, HID), lambda i: (0, 0)),
          pl.BlockSpec((1, HID), lambda i: (0, 0)),
          pl.BlockSpec((1, HID), lambda i: (0, 0)),
          pl.BlockSpec((HID, nxt), lambda i: (0, 0)),
      ],
      out_specs=[
          pl.BlockSpec((BLK, HID), lambda i: (i, 0)),
          pl.BlockSpec((BLK, nxt), lambda i: (i, 0)),
          pl.BlockSpec((BLK, 1), lambda i: (i, 0)),
      ],
      out_shape=[
          jax.ShapeDtypeStruct((N, HID), jnp.float32),
          jax.ShapeDtypeStruct((N, nxt), jnp.float32),
          jax.ShapeDtypeStruct((N, 1), jnp.float32),
      ],
  )(S, S, D, D, x, wl, wr, bb, g, be, wn)


def _b12_body(s0, s1, inv_ref, x_ref, wr, bb, g, be, wn, xo, yo):
  inv = inv_ref[...]
  h = (s0[...] + s1[...]) * inv + bb[...] + jnp.dot(
      x_ref[...], wr[...], preferred_element_type=jnp.float32)
  h = _gelu(_ln(h, g[...], be[...]))
  xn = h + x_ref[...]
  xo[...] = xn
  yo[...] = jnp.dot(xn, wn[...], preferred_element_type=jnp.float32)


def _tc_block(S, invd, x, wr, bb, g, be, wn):
  nxt = wn.shape[1]
  return pl.pallas_call(
      _b12_body,
      grid=(NBLK,),
      in_specs=[
          pl.BlockSpec((BLK, HID), lambda i: (i, 0)),
          pl.BlockSpec((BLK, HID), lambda i: (i + NBLK, 0)),
          pl.BlockSpec((BLK, 1), lambda i: (i, 0)),
          pl.BlockSpec((BLK, HID), lambda i: (i, 0)),
          pl.BlockSpec((HID, HID), lambda i: (0, 0)),
          pl.BlockSpec((1, HID), lambda i: (0, 0)),
          pl.BlockSpec((1, HID), lambda i: (0, 0)),
          pl.BlockSpec((1, HID), lambda i: (0, 0)),
          pl.BlockSpec((HID, nxt), lambda i: (0, 0)),
      ],
      out_specs=[
          pl.BlockSpec((BLK, HID), lambda i: (i, 0)),
          pl.BlockSpec((BLK, nxt), lambda i: (i, 0)),
      ],
      out_shape=[
          jax.ShapeDtypeStruct((N, HID), jnp.float32),
          jax.ShapeDtypeStruct((N, nxt), jnp.float32),
      ],
  )(S, S, invd, x, wr, bb, g, be, wn)


def _b3_body(s0, s1, inv_ref, x_ref, wr, bb, g, be, wc, bc, o_ref):
  inv = inv_ref[...]
  h = (s0[:, :OUT_HALF] + s1[:, :OUT_HALF]) * inv + bb[...] + jnp.dot(
      x_ref[...], wr[...], preferred_element_type=jnp.float32)
  h = _gelu(h)
  h = _ln(h, g[...], be[...])
  o_ref[...] = jnp.dot(h, wc[...], preferred_element_type=jnp.float32) + bc[...]


def _tc_head(S, invd, x, wr, bb, g, be, wc, bc):
  return pl.pallas_call(
      _b3_body,
      grid=(NBLK,),
      in_specs=[
          pl.BlockSpec((BLK, HID), lambda i: (i, 0)),
          pl.BlockSpec((BLK, HID), lambda i: (i + NBLK, 0)),
          pl.BlockSpec((BLK, 1), lambda i: (i, 0)),
          pl.BlockSpec((BLK, HID), lambda i: (i, 0)),
          pl.BlockSpec((HID, OUT_HALF), lambda i: (0, 0)),
          pl.BlockSpec((1, OUT_HALF), lambda i: (0, 0)),
          pl.BlockSpec((1, OUT_HALF), lambda i: (0, 0)),
          pl.BlockSpec((1, OUT_HALF), lambda i: (0, 0)),
          pl.BlockSpec((OUT_HALF, N_CLASSES), lambda i: (0, 0)),
          pl.BlockSpec((1, N_CLASSES), lambda i: (0, 0)),
      ],
      out_specs=pl.BlockSpec((BLK, N_CLASSES), lambda i: (i, 0)),
      out_shape=jax.ShapeDtypeStruct((N, N_CLASSES), jnp.float32),
  )(S, S, invd, x, wr, bb, g, be, wc, bc)


def kernel(x, edge_index, W1l, b1, W1r, ln1_g, ln1_b, W2l, b2, W2r,
           ln2_g, ln2_b, W3l, b3, W3r, lnc_g, lnc_b, Wc, bc):
  ei = edge_index.astype(jnp.int32)
  srcf = ei[0].reshape(NW, EPW)
  dstr = ei[1].reshape(NW, NCHUNK, C)
  zf = jnp.zeros((Z0, HID), jnp.float32)
  onesf = jnp.ones((C, HID), jnp.float32)
  W3lp = jnp.concatenate(
      [W3l, jnp.zeros((HID, HID - OUT_HALF), jnp.float32)], axis=1)

  r2 = lambda v: v.reshape(1, -1)

  S1, D = _seg1_deg(x, srcf, dstr, zf, onesf)
  x1, y2, invd = _tc_block1(S1, D, x, W1l, W1r, r2(b1), r2(ln1_g), r2(ln1_b),
                            W2l)
  S2 = _seg_128(y2, srcf, dstr, zf)
  x2, y3 = _tc_block(S2, invd, x1, W2r, r2(b2), r2(ln2_g), r2(ln2_b), W3lp)
  S3 = _seg_128(y3, srcf, dstr, zf)
  return _tc_head(S3, invd, x2, W3r, r2(b3), r2(lnc_g), r2(lnc_b),
                  Wc, r2(bc))
